# Initial kernel scaffold; baseline (speedup 1.0000x reference)
import jax
import jax.numpy as jnp
from jax.experimental import pallas as pl


def kernel(tokens, wte_weight, learned_embedding):
    raise NotImplementedError("write your pallas kernel here")


# speedup vs baseline: 7.8665x; 7.8665x over previous
import jax
import jax.numpy as jnp
from jax import lax
from jax.experimental import pallas as pl
from jax.experimental.pallas import tpu as pltpu, tpu_sc as plsc

VOCAB = 100000
EMBED_DIM = 64
BATCH = 16384
SEQ = 200
N_TOKENS = 20
SPLIT1 = 10

NC, NS, L = 2, 16, 16
NW = NC * NS
TOTAL = BATCH * SEQ
C = 800
ROWS_PER_W = TOTAL // NW
N_CHUNKS = ROWS_PER_W // C
P = 400
SUBS = (128, 128, 128, 128, 128, 128, 32)


def _body(tok_hbm, table_hbm, out_hbm, tok_v, idx_v, rows_v, lv_t, sem):
    wid = lax.axis_index("s") * NC + lax.axis_index("c")
    iota = lax.iota(jnp.int32, L)

    for g in range(P // L):
        j = (g * L + iota) % SEQ
        lv = jnp.where(
            (j >= 1) & (j <= SPLIT1), VOCAB + j - 1,
            jnp.where((j >= SPLIT1 + 2) & (j <= N_TOKENS + 1), VOCAB + j - 2,
                      -1))
        lv_t[pl.ds(g * L, L)] = lv

    def chunk(i, carry):
        r0 = wid * ROWS_PER_W + i * C
        pltpu.sync_copy(tok_hbm.at[pl.ds(r0, C)], tok_v)
        for g in range(C // L):
            t = (g * L) % P
            lv = lv_t[pl.ds(t, L)]
            idx = jnp.where(lv >= 0, lv, tok_v[pl.ds(g * L, L)])
            if any((g * L + l) % SEQ == SPLIT1 + 1 for l in range(L)):
                shifted = tok_v[pl.ds(g * L + (N_TOKENS - SPLIT1), L)]
                jvec = (g * L + iota) % SEQ
                idx = jnp.where(jvec == SPLIT1 + 1, shifted, idx)
            idx_v[pl.ds(g * L, L)] = idx
        copies = []
        off = 0
        for nsub in SUBS:
            copies.append(pltpu.async_copy(
                table_hbm.at[idx_v.at[pl.ds(off, nsub)]],
                rows_v.at[pl.ds(off, nsub)], sem))
            off += nsub
        for cp in copies:
            cp.wait()
        pltpu.sync_copy(rows_v, out_hbm.at[pl.ds(r0, C)])
        return carry

    lax.fori_loop(0, N_CHUNKS, chunk, 0)


def kernel(tokens, wte_weight, learned_embedding):
    table = jnp.concatenate([wte_weight, learned_embedding], axis=0)
    tok_flat = tokens.reshape(TOTAL).astype(jnp.int32)
    mesh = plsc.VectorSubcoreMesh(core_axis_name="c", subcore_axis_name="s",
                                  num_cores=NC, num_subcores=NS)
    out = pl.kernel(
        _body,
        out_type=jax.ShapeDtypeStruct((TOTAL, EMBED_DIM), jnp.float32),
        mesh=mesh,
        compiler_params=pltpu.CompilerParams(use_tc_tiling_on_sc=False),
        scratch_types=[
            pltpu.VMEM((C,), jnp.int32),
            pltpu.VMEM((C,), jnp.int32),
            pltpu.VMEM((C, EMBED_DIM), jnp.float32),
            pltpu.VMEM((P,), jnp.int32),
            pltpu.SemaphoreType.DMA,
        ],
    )(tok_flat, table)
    return out.reshape(BATCH, SEQ, EMBED_DIM)


# speedup vs baseline: 7.8811x; 1.0019x over previous
import jax
import jax.numpy as jnp
from jax import lax
from jax.experimental import pallas as pl
from jax.experimental.pallas import tpu as pltpu, tpu_sc as plsc

VOCAB = 100000
EMBED_DIM = 64
BATCH = 16384
SEQ = 200
N_TOKENS = 20
SPLIT1 = 10

NC, NS, L = 2, 16, 16
NW = NC * NS
TOTAL = BATCH * SEQ
C = 800
ROWS_PER_W = TOTAL // NW
N_CHUNKS = ROWS_PER_W // C
P = 400
SUBS = (128, 128, 128, 128, 128, 128, 32)


def _body(tok_hbm, table_hbm, out_hbm, tok_v, idx_v, rows_v, lv_t, sem):
    wid = lax.axis_index("s") * NC + lax.axis_index("c")
    iota = lax.iota(jnp.int32, L)

    for g in range(P // L):
        j = (g * L + iota) % SEQ
        lv = jnp.where(
            (j >= 1) & (j <= SPLIT1), VOCAB + j - 1,
            jnp.where((j >= SPLIT1 + 2) & (j <= N_TOKENS + 1), VOCAB + j - 2,
                      -1))
        lv_t[pl.ds(g * L, L)] = lv

    def chunk(i, carry):
        r0 = wid * ROWS_PER_W + i * C
        pltpu.sync_copy(tok_hbm.at[pl.ds(r0, C)], tok_v)
        for g in range(C // L):
            t = (g * L) % P
            lv = lv_t[pl.ds(t, L)]
            idx = jnp.where(lv >= 0, lv, tok_v[pl.ds(g * L, L)])
            if any((g * L + l) % SEQ == SPLIT1 + 1 for l in range(L)):
                shifted = tok_v[pl.ds(g * L + (N_TOKENS - SPLIT1), L)]
                jvec = (g * L + iota) % SEQ
                idx = jnp.where(jvec == SPLIT1 + 1, shifted, idx)
            idx_v[pl.ds(g * L, L)] = idx
        copies = []
        for m in range(C // SEQ):
            for off, nsub in ((0, 128), (128, SEQ - 128)):
                copies.append(pltpu.async_copy(
                    table_hbm.at[idx_v.at[pl.ds(m * SEQ + off, nsub)]],
                    rows_v.at[m, pl.ds(off, nsub)], sem))
        for cp in copies:
            cp.wait()
        b0 = wid * (ROWS_PER_W // SEQ) + i * (C // SEQ)
        pltpu.sync_copy(rows_v, out_hbm.at[pl.ds(b0, C // SEQ)])
        return carry

    lax.fori_loop(0, N_CHUNKS, chunk, 0)


def kernel(tokens, wte_weight, learned_embedding):
    table = jnp.concatenate([wte_weight, learned_embedding], axis=0)
    tok_flat = tokens.reshape(TOTAL).astype(jnp.int32)
    mesh = plsc.VectorSubcoreMesh(core_axis_name="c", subcore_axis_name="s",
                                  num_cores=NC, num_subcores=NS)
    out = pl.kernel(
        _body,
        out_type=jax.ShapeDtypeStruct((BATCH, SEQ, EMBED_DIM), jnp.float32),
        mesh=mesh,
        compiler_params=pltpu.CompilerParams(use_tc_tiling_on_sc=False),
        scratch_types=[
            pltpu.VMEM((C,), jnp.int32),
            pltpu.VMEM((C,), jnp.int32),
            pltpu.VMEM((C // SEQ, SEQ, EMBED_DIM), jnp.float32),
            pltpu.VMEM((P,), jnp.int32),
            pltpu.SemaphoreType.DMA,
        ],
    )(tok_flat, table)
    return out
